# baseline (speedup 1.0000x reference)
import functools
import math

import jax
import jax.numpy as jnp
from jax import lax
from jax.experimental import pallas as pl
from jax.experimental.pallas import tpu as pltpu, tpu_sc as plsc

V = 100000
N_ROWS = 1024
SMOOTH = 0.1
CONF = 1.0 - SMOOTH
EPS = SMOOTH / (V - 2)
K_ROW = CONF * math.log(CONF) + (V - 2) * EPS * math.log(EPS)

V_TC = 86688
VB = 3096


def _tc_body(tok_ref, xt_ref, out_ref):
    j = pl.program_id(0)
    c = tok_ref[...]
    notpad = (c != 0.0).astype(jnp.float32)
    xt = xt_ref[...]
    contrib = jnp.sum((-EPS * notpad) * xt)
    extra = jnp.sum(notpad * (K_ROW + EPS * xt[0:1, :]))
    contrib = contrib + jnp.where(j == 0, extra, 0.0)

    @pl.when(j == 0)
    def _init():
        out_ref[...] = jnp.zeros((1, 1), jnp.float32)

    out_ref[...] += jnp.full((1, 1), contrib, jnp.float32)


def _tc_pass(tok_row, xt):
    out = pl.pallas_call(
        _tc_body,
        grid=(V_TC // VB,),
        in_specs=[
            pl.BlockSpec((1, N_ROWS), lambda j: (0, 0)),
            pl.BlockSpec((VB, N_ROWS), lambda j: (j, 0)),
        ],
        out_specs=pl.BlockSpec((1, 1), lambda j: (0, 0)),
        out_shape=jax.ShapeDtypeStruct((1, 1), jnp.float32),
    )(tok_row, xt)
    return out[0, 0]


_SC_INFO = plsc.get_sparse_core_info()
_NC, _NS = _SC_INFO.num_cores, _SC_INFO.num_subcores
_NW = _NC * _NS
_BPW = N_ROWS // _NW
_V_SC = V - V_TC
_RPW = _V_SC // _NW
_NBANDS = _RPW // 16

_sc_mesh = plsc.VectorSubcoreMesh(core_axis_name="c", subcore_axis_name="s")


@functools.partial(
    pl.kernel,
    mesh=_sc_mesh,
    out_type=[
        jax.ShapeDtypeStruct((N_ROWS, 16), jnp.float32),
        jax.ShapeDtypeStruct((_NW, N_ROWS), jnp.float32),
    ],
    scratch_types=[
        pltpu.VMEM((_BPW,), jnp.int32),
        pltpu.VMEM((_BPW, N_ROWS), jnp.float32),
        pltpu.VMEM((_BPW, 16), jnp.float32),
        pltpu.VMEM((16, N_ROWS), jnp.float32),
        pltpu.VMEM((16, N_ROWS), jnp.float32),
        pltpu.VMEM((N_ROWS,), jnp.float32),
        pltpu.SemaphoreType.DMA,
        pltpu.SemaphoreType.DMA,
        pltpu.SemaphoreType.DMA,
    ],
)
def _sc_pass(xt_hbm, tok_hbm, g_hbm, cs_hbm,
             tok_v, rows_v, g_v, buf0, buf1, acc_v, gsem, sem0, sem1):
    wid = lax.axis_index("s") * _NC + lax.axis_index("c")
    base = wid * _BPW
    lane = lax.iota(jnp.int32, 16)

    pltpu.sync_copy(tok_hbm.at[pl.ds(base, _BPW)], tok_v)
    gather = pltpu.async_copy(xt_hbm.at[tok_v], rows_v, gsem)

    row0 = V_TC + wid * _RPW
    for cc in range(N_ROWS // 16):
        acc_v[pl.ds(16 * cc, 16)] = jnp.zeros((16,), jnp.float32)

    def chunk_copy(b, buf, sem):
        return pltpu.make_async_copy(
            xt_hbm.at[pl.ds(row0 + 16 * b, 16), :], buf, sem)

    chunk_copy(0, buf0, sem0).start()
    chunk_copy(1, buf1, sem1).start()

    def accumulate(buf):
        for sub in range(2):
            for cc in range(N_ROWS // 16):
                s = buf[8 * sub, pl.ds(16 * cc, 16)]
                for r in range(1, 8):
                    s += buf[8 * sub + r, pl.ds(16 * cc, 16)]
                acc_v[pl.ds(16 * cc, 16)] += s

    def loop_body(m, _):
        b = 2 * m
        pltpu.make_async_copy(
            xt_hbm.at[pl.ds(row0, 16), :], buf0, sem0).wait()
        accumulate(buf0)

        @pl.when(b + 2 < _NBANDS)
        def _():
            chunk_copy(b + 2, buf0, sem0).start()

        pltpu.make_async_copy(
            xt_hbm.at[pl.ds(row0, 16), :], buf1, sem1).wait()
        accumulate(buf1)

        @pl.when(b + 3 < _NBANDS)
        def _():
            chunk_copy(b + 3, buf1, sem1).start()

        return 0

    lax.fori_loop(0, _NBANDS // 2, loop_body, 0)
    pltpu.sync_copy(acc_v, cs_hbm.at[wid])

    gather.wait()
    for k in range(_BPW):
        i_col = base + k
        vec = rows_v[k, pl.ds((i_col // 16) * 16, 16)]
        g_v.at[k][...] = jnp.where(lane == i_col % 16, vec, 0.0)
    pltpu.sync_copy(g_v, g_hbm.at[pl.ds(base, _BPW)])


def kernel(predicted_log_probabilities, tgt_tokens):
    n, v = predicted_log_probabilities.shape
    xt = predicted_log_probabilities.T
    tok_row = tgt_tokens.reshape(1, n).astype(jnp.float32)
    g16, cs = _sc_pass(xt, tgt_tokens)
    a = _tc_pass(tok_row, xt)
    notpad = (tgt_tokens != 0).astype(jnp.float32)
    g = jnp.sum(g16, axis=1)
    colsum = jnp.sum(cs, axis=0)
    return (a
            - EPS * jnp.sum(notpad * colsum)
            - (CONF - EPS) * jnp.sum(notpad * g))
